# Initial kernel scaffold; baseline (speedup 1.0000x reference)
import jax
import jax.numpy as jnp
from jax.experimental import pallas as pl


def kernel(text, text_lengths, table, W1, b1, W2, b2):
    raise NotImplementedError("write your pallas kernel here")


# speedup vs baseline: 10.8515x; 10.8515x over previous
import functools

import jax
import jax.numpy as jnp
from jax import lax
from jax.experimental import pallas as pl
from jax.experimental.pallas import tpu as pltpu
from jax.experimental.pallas import tpu_sc as plsc

_NP = 16


def _proj_body(table_ref, w1_ref, w2p_ref, out_ref, *, scale):
    r = lax.dot_general(w2p_ref[...], w1_ref[...], (((1,), (0,)), ((), ())),
                        precision=lax.Precision.HIGHEST,
                        preferred_element_type=jnp.float32) * scale
    res = lax.dot_general(table_ref[...], r, (((1,), (1,)), ((), ())),
                          precision=lax.Precision.HIGHEST,
                          preferred_element_type=jnp.float32)
    rows = lax.broadcasted_iota(jnp.int32, res.shape, 0)
    res = jnp.where((rows == 0) & (pl.program_id(0) == 0), 0.0, res)
    out_ref[...] = res


def _project_table(table, w1, w2p, scale):
    v, h = table.shape
    blk = 5000 if v % 5000 == 0 else 8
    grid = (v // blk,)
    return pl.pallas_call(
        functools.partial(_proj_body, scale=scale),
        grid=grid,
        in_specs=[
            pl.BlockSpec((blk, h), lambda i: (i, 0)),
            pl.BlockSpec((h, h), lambda i: (0, 0)),
            pl.BlockSpec((_NP, h), lambda i: (0, 0)),
        ],
        out_specs=pl.BlockSpec((blk, _NP), lambda i: (i, 0)),
        out_shape=jax.ShapeDtypeStruct((v, _NP), jnp.float32),
    )(table, w1, w2p)


def _gather_sum(text_flat, tq, batch, seq):
    info = plsc.get_sparse_core_info()
    ncores, nsub = info.num_cores, info.num_subcores
    nw = ncores * nsub
    bpw = batch // nw
    chunks = []
    off = 0
    while off < seq:
        sz = min(128, seq - off)
        chunks.append((off, sz))
        off += sz

    mesh = plsc.VectorSubcoreMesh(core_axis_name="c", subcore_axis_name="s")

    @functools.partial(
        pl.kernel,
        mesh=mesh,
        compiler_params=pltpu.CompilerParams(use_tc_tiling_on_sc=False),
        out_type=jax.ShapeDtypeStruct((batch, _NP), jnp.float32),
        scratch_types=[
            pltpu.VMEM((bpw * seq,), jnp.int32),
            pltpu.VMEM((2, seq, _NP), jnp.float32),
            pltpu.VMEM((bpw, _NP), jnp.float32),
            pltpu.SemaphoreType.DMA,
            pltpu.SemaphoreType.DMA,
        ],
    )
    def k(text_hbm, tq_hbm, out_hbm, idx_v, buf_v, out_v, sem0, sem1):
        sems = (sem0, sem1)
        wid = lax.axis_index("s") * ncores + lax.axis_index("c")
        base = wid * (bpw * seq)
        pltpu.sync_copy(text_hbm.at[pl.ds(base, bpw * seq)], idx_v)

        def issue(s, b):
            for (o, sz) in chunks:
                pltpu.async_copy(
                    tq_hbm.at[idx_v.at[pl.ds(s * seq + o, sz)]],
                    buf_v.at[b, pl.ds(o, sz)],
                    sems[b])

        def wait(b):
            for (o, sz) in chunks:
                pltpu.make_async_copy(
                    tq_hbm.at[pl.ds(0, sz)],
                    buf_v.at[b, pl.ds(o, sz)],
                    sems[b]).wait()

        def accum(s, b):
            zero = jnp.zeros((_NP,), jnp.float32)

            def body(l, accs):
                a0, a1, a2, a3 = accs
                r = l * 4
                return (a0 + buf_v[b, r, :], a1 + buf_v[b, r + 1, :],
                        a2 + buf_v[b, r + 2, :], a3 + buf_v[b, r + 3, :])

            a0, a1, a2, a3 = lax.fori_loop(0, seq // 4, body,
                                           (zero, zero, zero, zero))
            out_v[s, :] = (a0 + a1) + (a2 + a3)

        issue(0, 0)

        def body(g, _):
            s0 = g * 2
            issue(s0 + 1, 1)
            wait(0)
            accum(s0, 0)

            @pl.when(s0 + 2 < bpw)
            def _():
                issue(s0 + 2, 0)

            wait(1)
            accum(s0 + 1, 1)
            return 0

        lax.fori_loop(0, bpw // 2, body, 0)
        pltpu.sync_copy(out_v, out_hbm.at[pl.ds(wid * bpw, bpw)])

    return k(text_flat, tq)


def _finish_body(z_ref, w2p_ref, b1_ref, b2p_ref, out_ref, *, ncls):
    c = lax.dot_general(b1_ref[...], w2p_ref[...], (((1,), (1,)), ((), ())),
                        precision=lax.Precision.HIGHEST,
                        preferred_element_type=jnp.float32) + b2p_ref[...]
    z = z_ref[...] + c
    zs = z[:, :ncls]
    m = jnp.max(zs, axis=1, keepdims=True)
    e = jnp.exp(zs - m)
    out_ref[...] = (zs - m) - jnp.log(jnp.sum(e, axis=1, keepdims=True))


def _finish(zacc, w2p, b1, b2p, ncls):
    batch = zacc.shape[0]
    return pl.pallas_call(
        functools.partial(_finish_body, ncls=ncls),
        in_specs=[
            pl.BlockSpec(zacc.shape, lambda: (0, 0)),
            pl.BlockSpec(w2p.shape, lambda: (0, 0)),
            pl.BlockSpec((1, b1.shape[0]), lambda: (0, 0)),
            pl.BlockSpec((1, _NP), lambda: (0, 0)),
        ],
        out_specs=pl.BlockSpec((batch, ncls), lambda: (0, 0)),
        out_shape=jax.ShapeDtypeStruct((batch, ncls), jnp.float32),
    )(zacc, w2p, b1.reshape(1, -1), b2p.reshape(1, -1))


def kernel(text, text_lengths, table, W1, b1, W2, b2):
    del text_lengths
    batch, seq = text.shape
    ncls, h = W2.shape
    w2p = jnp.zeros((_NP, h), W2.dtype).at[:ncls].set(W2)
    b2p = jnp.zeros((_NP,), b2.dtype).at[:ncls].set(b2)
    tq = _project_table(table, W1, w2p, 1.0 / seq)
    zacc = _gather_sum(text.reshape(-1), tq, batch, seq)
    return _finish(zacc, w2p, b1, b2p, ncls)


# speedup vs baseline: 11.9547x; 1.1017x over previous
import functools

import jax
import jax.numpy as jnp
from jax import lax
from jax.experimental import pallas as pl
from jax.experimental.pallas import tpu as pltpu
from jax.experimental.pallas import tpu_sc as plsc

_NP = 16


def _proj_body(table_ref, w1_ref, w2p_ref, out_ref, *, scale):
    r = lax.dot_general(w2p_ref[...], w1_ref[...], (((1,), (0,)), ((), ())),
                        precision=lax.Precision.HIGHEST,
                        preferred_element_type=jnp.float32) * scale
    res = lax.dot_general(table_ref[...], r, (((1,), (1,)), ((), ())),
                          precision=lax.Precision.HIGHEST,
                          preferred_element_type=jnp.float32)
    rows = lax.broadcasted_iota(jnp.int32, res.shape, 0)
    res = jnp.where((rows == 0) & (pl.program_id(0) == 0), 0.0, res)
    out_ref[...] = jnp.concatenate(
        [res, jnp.zeros((res.shape[0], 128 - _NP), jnp.float32)], axis=1)


def _project_table(table, w1, w2p, scale):
    v, h = table.shape
    blk = 5000 if v % 5000 == 0 else 8
    grid = (v // blk,)
    return pl.pallas_call(
        functools.partial(_proj_body, scale=scale),
        grid=grid,
        in_specs=[
            pl.BlockSpec((blk, h), lambda i: (i, 0)),
            pl.BlockSpec((h, h), lambda i: (0, 0)),
            pl.BlockSpec((_NP, h), lambda i: (0, 0)),
        ],
        out_specs=pl.BlockSpec((blk, 128), lambda i: (i, 0)),
        out_shape=jax.ShapeDtypeStruct((v, 128), jnp.float32),
    )(table, w1, w2p)


def _gather_sum(text_flat, tq, batch, seq):
    info = plsc.get_sparse_core_info()
    ncores, nsub = info.num_cores, info.num_subcores
    nw = ncores * nsub
    bpw = batch // nw
    chunks = []
    off = 0
    while off < seq:
        sz = min(128, seq - off)
        chunks.append((off, sz))
        off += sz

    mesh = plsc.VectorSubcoreMesh(core_axis_name="c", subcore_axis_name="s")

    @functools.partial(
        pl.kernel,
        mesh=mesh,
        compiler_params=pltpu.CompilerParams(use_tc_tiling_on_sc=False),
        out_type=jax.ShapeDtypeStruct((batch, _NP), jnp.float32),
        scratch_types=[
            pltpu.VMEM((bpw * seq,), jnp.int32),
            pltpu.VMEM((2, seq, _NP), jnp.float32),
            pltpu.VMEM((bpw, _NP), jnp.float32),
            pltpu.SemaphoreType.DMA,
            pltpu.SemaphoreType.DMA,
        ],
    )
    def k(text_hbm, tq_hbm, out_hbm, idx_v, buf_v, out_v, sem0, sem1):
        sems = (sem0, sem1)
        wid = lax.axis_index("s") * ncores + lax.axis_index("c")
        base = wid * (bpw * seq)
        pltpu.sync_copy(text_hbm.at[pl.ds(base, bpw * seq)], idx_v)

        def scale_idx(i, _):
            idx_v[pl.ds(i * 16, 16)] = idx_v[pl.ds(i * 16, 16)] * 8
            return 0

        lax.fori_loop(0, (bpw * seq) // 16, scale_idx, 0)

        def issue(s, b):
            for (o, sz) in chunks:
                pltpu.async_copy(
                    tq_hbm.at[idx_v.at[pl.ds(s * seq + o, sz)]],
                    buf_v.at[b, pl.ds(o, sz)],
                    sems[b])

        def wait(b):
            for (o, sz) in chunks:
                pltpu.make_async_copy(
                    tq_hbm.at[pl.ds(0, sz)],
                    buf_v.at[b, pl.ds(o, sz)],
                    sems[b]).wait()

        def accum(s, b):
            zero = jnp.zeros((_NP,), jnp.float32)

            def body(l, accs):
                a0, a1, a2, a3 = accs
                r = l * 4
                return (a0 + buf_v[b, r, :], a1 + buf_v[b, r + 1, :],
                        a2 + buf_v[b, r + 2, :], a3 + buf_v[b, r + 3, :])

            a0, a1, a2, a3 = lax.fori_loop(0, seq // 4, body,
                                           (zero, zero, zero, zero))
            out_v[s, :] = (a0 + a1) + (a2 + a3)

        issue(0, 0)

        def body(g, _):
            s0 = g * 2
            issue(s0 + 1, 1)
            wait(0)
            accum(s0, 0)

            @pl.when(s0 + 2 < bpw)
            def _():
                issue(s0 + 2, 0)

            wait(1)
            accum(s0 + 1, 1)
            return 0

        lax.fori_loop(0, bpw // 2, body, 0)
        pltpu.sync_copy(out_v, out_hbm.at[pl.ds(wid * bpw, bpw)])

    return k(text_flat, tq)


def _finish_body(z_ref, w2p_ref, b1_ref, b2p_ref, out_ref, *, ncls):
    c = lax.dot_general(b1_ref[...], w2p_ref[...], (((1,), (1,)), ((), ())),
                        precision=lax.Precision.HIGHEST,
                        preferred_element_type=jnp.float32) + b2p_ref[...]
    z = z_ref[...] + c
    zs = z[:, :ncls]
    m = jnp.max(zs, axis=1, keepdims=True)
    e = jnp.exp(zs - m)
    out_ref[...] = (zs - m) - jnp.log(jnp.sum(e, axis=1, keepdims=True))


def _finish(zacc, w2p, b1, b2p, ncls):
    batch = zacc.shape[0]
    return pl.pallas_call(
        functools.partial(_finish_body, ncls=ncls),
        in_specs=[
            pl.BlockSpec(zacc.shape, lambda: (0, 0)),
            pl.BlockSpec(w2p.shape, lambda: (0, 0)),
            pl.BlockSpec((1, b1.shape[0]), lambda: (0, 0)),
            pl.BlockSpec((1, _NP), lambda: (0, 0)),
        ],
        out_specs=pl.BlockSpec((batch, ncls), lambda: (0, 0)),
        out_shape=jax.ShapeDtypeStruct((batch, ncls), jnp.float32),
    )(zacc, w2p, b1.reshape(1, -1), b2p.reshape(1, -1))


def kernel(text, text_lengths, table, W1, b1, W2, b2):
    del text_lengths
    batch, seq = text.shape
    ncls, h = W2.shape
    w2p = jnp.zeros((_NP, h), W2.dtype).at[:ncls].set(W2)
    b2p = jnp.zeros((_NP,), b2.dtype).at[:ncls].set(b2)
    tq = _project_table(table, W1, w2p, 1.0 / seq)
    tq8 = tq.reshape(-1, _NP)
    zacc = _gather_sum(text.reshape(-1), tq8, batch, seq)
    return _finish(zacc, w2p, b1, b2p, ncls)


# speedup vs baseline: 16.3937x; 1.3713x over previous
import functools

import jax
import jax.numpy as jnp
from jax import lax
from jax.experimental import pallas as pl
from jax.experimental.pallas import tpu as pltpu
from jax.experimental.pallas import tpu_sc as plsc

_NP = 16


_PACK = 128 // _NP


def _proj_body(t8_ref, w1_ref, w2p_ref, out_ref, *, scale, h):
    rt = lax.dot_general(w1_ref[...], w2p_ref[...], (((0,), (1,)), ((), ())),
                         preferred_element_type=jnp.float32) * scale
    wbig = jnp.tile(rt, (_PACK, _PACK))
    rows = lax.broadcasted_iota(jnp.int32, wbig.shape, 0) // h
    cols = lax.broadcasted_iota(jnp.int32, wbig.shape, 1) // _NP
    wbig = jnp.where(rows == cols, wbig, 0.0)
    res = lax.dot_general(t8_ref[...], wbig, (((1,), (0,)), ((), ())),
                          preferred_element_type=jnp.float32)
    rid = lax.broadcasted_iota(jnp.int32, res.shape, 0)
    cid = lax.broadcasted_iota(jnp.int32, res.shape, 1)
    res = jnp.where((rid == 0) & (pl.program_id(0) == 0) & (cid < _NP),
                    0.0, res)
    out_ref[...] = res


def _project_table(table, w1, w2p, scale):
    v, h = table.shape
    vp = v // _PACK
    t8 = table.reshape(vp, _PACK * h)
    blk = 3200
    nsteps = (vp + blk - 1) // blk
    return pl.pallas_call(
        functools.partial(_proj_body, scale=scale, h=h),
        grid=(nsteps,),
        in_specs=[
            pl.BlockSpec((blk, _PACK * h), lambda i: (i, 0)),
            pl.BlockSpec((h, h), lambda i: (0, 0)),
            pl.BlockSpec((_NP, h), lambda i: (0, 0)),
        ],
        out_specs=pl.BlockSpec((blk, 128), lambda i: (i, 0)),
        out_shape=jax.ShapeDtypeStruct((vp, 128), jnp.float32),
    )(t8, w1, w2p)


def _gather_sum(text_flat, tq, batch, seq):
    info = plsc.get_sparse_core_info()
    ncores, nsub = info.num_cores, info.num_subcores
    nw = ncores * nsub
    bpw = batch // nw
    chunks = []
    off = 0
    while off < seq:
        sz = min(128, seq - off)
        chunks.append((off, sz))
        off += sz

    mesh = plsc.VectorSubcoreMesh(core_axis_name="c", subcore_axis_name="s")

    @functools.partial(
        pl.kernel,
        mesh=mesh,
        compiler_params=pltpu.CompilerParams(use_tc_tiling_on_sc=False),
        out_type=jax.ShapeDtypeStruct((batch, _NP), jnp.float32),
        scratch_types=[
            pltpu.VMEM((bpw * seq,), jnp.int32),
            pltpu.VMEM((2, seq, _NP), jnp.float32),
            pltpu.VMEM((bpw, _NP), jnp.float32),
            pltpu.SemaphoreType.DMA,
            pltpu.SemaphoreType.DMA,
        ],
    )
    def k(text_hbm, tq_hbm, out_hbm, idx_v, buf_v, out_v, sem0, sem1):
        sems = (sem0, sem1)
        wid = lax.axis_index("s") * ncores + lax.axis_index("c")
        base = wid * (bpw * seq)
        pltpu.sync_copy(text_hbm.at[pl.ds(base, bpw * seq)], idx_v)

        def issue(s, b):
            for (o, sz) in chunks:
                pltpu.async_copy(
                    tq_hbm.at[idx_v.at[pl.ds(s * seq + o, sz)]],
                    buf_v.at[b, pl.ds(o, sz)],
                    sems[b])

        def wait(b):
            for (o, sz) in chunks:
                pltpu.make_async_copy(
                    tq_hbm.at[pl.ds(0, sz)],
                    buf_v.at[b, pl.ds(o, sz)],
                    sems[b]).wait()

        def accum(s, b):
            zero = jnp.zeros((_NP,), jnp.float32)

            def body(l, accs):
                a0, a1, a2, a3 = accs
                r = l * 4
                return (a0 + buf_v[b, r, :], a1 + buf_v[b, r + 1, :],
                        a2 + buf_v[b, r + 2, :], a3 + buf_v[b, r + 3, :])

            a0, a1, a2, a3 = lax.fori_loop(0, seq // 4, body,
                                           (zero, zero, zero, zero))
            out_v[s, :] = (a0 + a1) + (a2 + a3)

        issue(0, 0)

        def body(g, _):
            s0 = g * 2
            issue(s0 + 1, 1)
            wait(0)
            accum(s0, 0)

            @pl.when(s0 + 2 < bpw)
            def _():
                issue(s0 + 2, 0)

            wait(1)
            accum(s0 + 1, 1)
            return 0

        lax.fori_loop(0, bpw // 2, body, 0)
        pltpu.sync_copy(out_v, out_hbm.at[pl.ds(wid * bpw, bpw)])

    return k(text_flat, tq)


def _finish_body(z_ref, w2p_ref, b1_ref, b2p_ref, out_ref, *, ncls):
    c = lax.dot_general(b1_ref[...], w2p_ref[...], (((1,), (1,)), ((), ())),
                        precision=lax.Precision.HIGHEST,
                        preferred_element_type=jnp.float32) + b2p_ref[...]
    z = z_ref[...] + c
    zs = z[:, :ncls]
    m = jnp.max(zs, axis=1, keepdims=True)
    e = jnp.exp(zs - m)
    out_ref[...] = (zs - m) - jnp.log(jnp.sum(e, axis=1, keepdims=True))


def _finish(zacc, w2p, b1, b2p, ncls):
    batch = zacc.shape[0]
    return pl.pallas_call(
        functools.partial(_finish_body, ncls=ncls),
        in_specs=[
            pl.BlockSpec(zacc.shape, lambda: (0, 0)),
            pl.BlockSpec(w2p.shape, lambda: (0, 0)),
            pl.BlockSpec((1, b1.shape[0]), lambda: (0, 0)),
            pl.BlockSpec((1, _NP), lambda: (0, 0)),
        ],
        out_specs=pl.BlockSpec((batch, ncls), lambda: (0, 0)),
        out_shape=jax.ShapeDtypeStruct((batch, ncls), jnp.float32),
    )(zacc, w2p, b1.reshape(1, -1), b2p.reshape(1, -1))


def kernel(text, text_lengths, table, W1, b1, W2, b2):
    del text_lengths
    batch, seq = text.shape
    ncls, h = W2.shape
    w2p = jnp.zeros((_NP, h), W2.dtype).at[:ncls].set(W2)
    b2p = jnp.zeros((_NP,), b2.dtype).at[:ncls].set(b2)
    tq = _project_table(table, W1, w2p, 1.0 / seq)
    tq8 = tq.reshape(-1, _NP)
    zacc = _gather_sum(text.reshape(-1), tq8, batch, seq)
    return _finish(zacc, w2p, b1, b2p, ncls)


# speedup vs baseline: 17.1208x; 1.0444x over previous
import functools

import jax
import jax.numpy as jnp
from jax import lax
from jax.experimental import pallas as pl
from jax.experimental.pallas import tpu as pltpu
from jax.experimental.pallas import tpu_sc as plsc

_NP = 16


_PACK = 128 // _NP


def _proj_body(t_ref, w1_ref, w2p_ref, out_ref, *, scale, h):
    rt = lax.dot_general(w1_ref[...], w2p_ref[...], (((0,), (1,)), ((), ())),
                         preferred_element_type=jnp.float32) * scale
    tbl = t_ref[...]
    n = tbl.shape[0]
    tp = jnp.concatenate(
        [tbl, jnp.zeros((n, 128 - h), jnp.float32)], axis=1)
    t8 = tp.reshape(n // _PACK, _PACK * 128)
    rtp = jnp.concatenate(
        [rt, jnp.zeros((128 - h, _NP), jnp.float32)], axis=0)
    wbig = jnp.tile(rtp, (_PACK, _PACK))
    rows = lax.broadcasted_iota(jnp.int32, wbig.shape, 0) // 128
    cols = lax.broadcasted_iota(jnp.int32, wbig.shape, 1) // _NP
    wbig = jnp.where(rows == cols, wbig, 0.0)
    res = lax.dot_general(t8, wbig, (((1,), (0,)), ((), ())),
                          preferred_element_type=jnp.float32)
    rid = lax.broadcasted_iota(jnp.int32, res.shape, 0)
    cid = lax.broadcasted_iota(jnp.int32, res.shape, 1)
    res = jnp.where((rid == 0) & (pl.program_id(0) == 0) & (cid < _NP),
                    0.0, res)
    out_ref[...] = res


def _project_table(table, w1, w2p, scale):
    v, h = table.shape
    vp = v // _PACK
    blk = 1600
    nsteps = (vp + blk - 1) // blk
    return pl.pallas_call(
        functools.partial(_proj_body, scale=scale, h=h),
        grid=(nsteps,),
        in_specs=[
            pl.BlockSpec((blk * _PACK, h), lambda i: (i, 0)),
            pl.BlockSpec((h, h), lambda i: (0, 0)),
            pl.BlockSpec((_NP, h), lambda i: (0, 0)),
        ],
        out_specs=pl.BlockSpec((blk, 128), lambda i: (i, 0)),
        out_shape=jax.ShapeDtypeStruct((vp, 128), jnp.float32),
    )(table, w1, w2p)


def _gather_sum(text_flat, tq, batch, seq):
    info = plsc.get_sparse_core_info()
    ncores, nsub = info.num_cores, info.num_subcores
    nw = ncores * nsub
    bpw = batch // nw
    chunks = []
    off = 0
    while off < seq:
        sz = min(128, seq - off)
        chunks.append((off, sz))
        off += sz

    mesh = plsc.VectorSubcoreMesh(core_axis_name="c", subcore_axis_name="s")

    @functools.partial(
        pl.kernel,
        mesh=mesh,
        compiler_params=pltpu.CompilerParams(use_tc_tiling_on_sc=False),
        out_type=jax.ShapeDtypeStruct((batch, _NP), jnp.float32),
        scratch_types=[
            pltpu.VMEM((bpw * seq,), jnp.int32),
            pltpu.VMEM((2, seq, _NP), jnp.float32),
            pltpu.VMEM((bpw, _NP), jnp.float32),
            pltpu.SemaphoreType.DMA,
            pltpu.SemaphoreType.DMA,
        ],
    )
    def k(text_hbm, tq_hbm, out_hbm, idx_v, buf_v, out_v, sem0, sem1):
        sems = (sem0, sem1)
        wid = lax.axis_index("s") * ncores + lax.axis_index("c")
        base = wid * (bpw * seq)
        pltpu.sync_copy(text_hbm.at[pl.ds(base, bpw * seq)], idx_v)

        def issue(s, b):
            for (o, sz) in chunks:
                pltpu.async_copy(
                    tq_hbm.at[idx_v.at[pl.ds(s * seq + o, sz)]],
                    buf_v.at[b, pl.ds(o, sz)],
                    sems[b])

        def wait(b):
            for (o, sz) in chunks:
                pltpu.make_async_copy(
                    tq_hbm.at[pl.ds(0, sz)],
                    buf_v.at[b, pl.ds(o, sz)],
                    sems[b]).wait()

        def accum(s, b):
            zero = jnp.zeros((_NP,), jnp.float32)

            def body(l, accs):
                a0, a1, a2, a3 = accs
                r = l * 4
                return (a0 + buf_v[b, r, :], a1 + buf_v[b, r + 1, :],
                        a2 + buf_v[b, r + 2, :], a3 + buf_v[b, r + 3, :])

            a0, a1, a2, a3 = lax.fori_loop(0, seq // 4, body,
                                           (zero, zero, zero, zero))
            out_v[s, :] = (a0 + a1) + (a2 + a3)

        issue(0, 0)

        def body(g, _):
            s0 = g * 2
            issue(s0 + 1, 1)
            wait(0)
            accum(s0, 0)

            @pl.when(s0 + 2 < bpw)
            def _():
                issue(s0 + 2, 0)

            wait(1)
            accum(s0 + 1, 1)
            return 0

        lax.fori_loop(0, bpw // 2, body, 0)
        pltpu.sync_copy(out_v, out_hbm.at[pl.ds(wid * bpw, bpw)])

    return k(text_flat, tq)


def _finish_body(z_ref, w2p_ref, b1_ref, b2p_ref, out_ref, *, ncls):
    c = lax.dot_general(b1_ref[...], w2p_ref[...], (((1,), (1,)), ((), ())),
                        precision=lax.Precision.HIGHEST,
                        preferred_element_type=jnp.float32) + b2p_ref[...]
    z = z_ref[...] + c
    zs = z[:, :ncls]
    m = jnp.max(zs, axis=1, keepdims=True)
    e = jnp.exp(zs - m)
    out_ref[...] = (zs - m) - jnp.log(jnp.sum(e, axis=1, keepdims=True))


def _finish(zacc, w2p, b1, b2p, ncls):
    batch = zacc.shape[0]
    return pl.pallas_call(
        functools.partial(_finish_body, ncls=ncls),
        in_specs=[
            pl.BlockSpec(zacc.shape, lambda: (0, 0)),
            pl.BlockSpec(w2p.shape, lambda: (0, 0)),
            pl.BlockSpec((1, b1.shape[0]), lambda: (0, 0)),
            pl.BlockSpec((1, _NP), lambda: (0, 0)),
        ],
        out_specs=pl.BlockSpec((batch, ncls), lambda: (0, 0)),
        out_shape=jax.ShapeDtypeStruct((batch, ncls), jnp.float32),
    )(zacc, w2p, b1.reshape(1, -1), b2p.reshape(1, -1))


def kernel(text, text_lengths, table, W1, b1, W2, b2):
    del text_lengths
    batch, seq = text.shape
    ncls, h = W2.shape
    w2p = jnp.zeros((_NP, h), W2.dtype).at[:ncls].set(W2)
    b2p = jnp.zeros((_NP,), b2.dtype).at[:ncls].set(b2)
    tq = _project_table(table, W1, w2p, 1.0 / seq)
    tq8 = tq.reshape(-1, _NP)
    zacc = _gather_sum(text.reshape(-1), tq8, batch, seq)
    return _finish(zacc, w2p, b1, b2p, ncls)
